# baseline (speedup 1.0000x reference)
import functools

import jax
import jax.numpy as jnp
from jax import lax
from jax.experimental import pallas as pl
from jax.experimental.pallas import tpu as pltpu
from jax.experimental.pallas import tpu_sc as plsc

N_POINTS = 3600
N_INT = 2000
R_BODY = 10.0
V0 = 1540.0
X0 = -12.0
DX = 1.6
Y0 = -12.0
DY = 1.6

L = 16
NC = 2
NS = 16
NW = NC * NS
N_GROUPS = N_POINTS // L
N_GROUPS_PAD = N_GROUPS + 1
G_MAX = 8
EPS = 1e-12

O_AX = 4096
O_AY = 4112
O_CX = 4128
O_CY = 4400
O_TBL = 4672
O_CNT = 4928
O_NZ = 4944
CONSTS_LEN = 5200
PARAMS_LEN = (N_GROUPS + 1) * 3 * 16


def _sc_body(in_hbm, wf_hbm, consts_v, params_v, yint_v, xint_v, out_v, sem1,
             sem2):
    wid = lax.axis_index("s") * NC + lax.axis_index("c")
    start_g = 7 * wid + jnp.minimum(wid, 1)
    cp1 = pltpu.async_copy(in_hbm.at[pl.ds(PARAMS_LEN, CONSTS_LEN)], consts_v,
                           sem1)
    cp2 = pltpu.async_copy(
        in_hbm.at[pl.ds(start_g * (3 * L), G_MAX * 3 * L)], params_v, sem2)
    cp1.wait()
    cp2.wait()
    tbl_v = consts_v.at[pl.ds(O_TBL, 256)]
    axv = consts_v[pl.ds(O_AX, L)]
    ayv = consts_v[pl.ds(O_AY, L)]
    iota = jax.lax.iota(jnp.int32, L)
    iota2 = iota + 16 * L
    n_w = jnp.where(wid == 0, G_MAX, 7)
    n_quads = (jnp.max(consts_v[pl.ds(O_CNT, L)]).astype(jnp.int32) + 3) >> 2

    def gather_at(xf, yf):
        xi = xf.astype(jnp.int32)
        row = yf.astype(jnp.int32) & 15
        return plsc.load_gather(tbl_v, [(row << 4) + xi])

    def run_group(i, _):
        off = i * (3 * L)
        bx = plsc.load_gather(params_v, [off + iota])
        by = plsc.load_gather(params_v, [off + L + iota])
        sc = plsc.load_gather(params_v, [off + 2 * L + iota])

        def boundaries(b, c_base):
            bp = jnp.where(b >= 0, jnp.maximum(b, EPS), jnp.minimum(b, -EPS))
            inv = 1.0 / bp
            bpos = bp > 0.0

            def q_at(m):
                t = consts_v[pl.ds(c_base + m * L, L)] * inv
                t = jnp.clip(t, -1.0, float(N_INT + 1))
                ti = t.astype(jnp.int32)
                tf = ti.astype(jnp.float32)
                up = jnp.where(jnp.where(bpos, t > tf, t >= tf), 1, 0)
                return jnp.clip(ti + up, 0, N_INT)

            q_prev = q_at(0)
            los, hips = [], []
            for m in range(1, 17):
                q_cur = q_at(m)
                los.append(jnp.minimum(q_prev, q_cur))
                hips.append(jnp.maximum(q_prev, q_cur))
                q_prev = q_cur
            return los, hips

        ylos, yhips = boundaries(by, O_CY)
        for k in range(16):
            yint_v[pl.ds(k * L, L)] = ylos[k]
            yint_v[pl.ds((16 + k) * L, L)] = yhips[k]
        xlos, xhips = boundaries(bx, O_CX)
        for m in range(16):
            xint_v[pl.ds(m * L, L)] = xlos[m]
            xint_v[pl.ds((16 + m) * L, L)] = xhips[m]

        def quad(qi, accs):
            accs = list(accs)
            for u in range(4):
                pc = plsc.bitcast(
                    plsc.load_gather(
                        consts_v,
                        [jnp.zeros((L,), jnp.int32) + (O_NZ + qi * 4 + u)]),
                    jnp.int32)
                xo = pc & 0xFF
                yo = (pc >> 8) & 0xFF
                tvo = pc >> 16
                xlo = plsc.load_gather(xint_v, [xo + iota])
                xhi = plsc.load_gather(xint_v, [xo + iota2])
                ylo = plsc.load_gather(yint_v, [yo + iota])
                yhi = plsc.load_gather(yint_v, [yo + iota2])
                tv = plsc.load_gather(consts_v, [tvo + iota])
                cnt = jnp.minimum(xhi, yhi) - jnp.maximum(xlo, ylo)
                cnt = jnp.maximum(cnt, 0).astype(jnp.float32)
                accs[u] = accs[u] + cnt * tv
            return tuple(accs)

        accs = lax.fori_loop(0, n_quads, quad,
                             tuple(jnp.zeros((L,), jnp.float32)
                                   for _ in range(4)))
        acc = (accs[0] + accs[1]) + (accs[2] + accs[3])

        v_first = gather_at(axv, ayv)
        v_last = gather_at(axv + bx * float(N_INT - 1),
                           ayv + by * float(N_INT - 1))
        wf = sc * (2.0 * acc - v_first - v_last)
        plsc.store_scatter(out_v, [i * L + iota], wf)
        return 0

    lax.fori_loop(0, n_w, run_group, 0)

    @pl.when(wid == 0)
    def _():
        pltpu.sync_copy(out_v, wf_hbm.at[pl.ds(0, G_MAX * L)])

    @pl.when(wid > 0)
    def _():
        pltpu.sync_copy(out_v.at[pl.ds(0, 7 * L)],
                        wf_hbm.at[pl.ds(G_MAX * L + (wid - 1) * 7 * L, 7 * L)])


@functools.cache
def _sc_integrate():
    return pl.kernel(
        _sc_body,
        out_type=jax.ShapeDtypeStruct((N_POINTS,), jnp.float32),
        mesh=plsc.VectorSubcoreMesh(core_axis_name="c", subcore_axis_name="s"),
        compiler_params=pltpu.CompilerParams(needs_layout_passes=False),
        scratch_types=[
            pltpu.VMEM((CONSTS_LEN,), jnp.float32),
            pltpu.VMEM((G_MAX * 3 * L,), jnp.float32),
            pltpu.VMEM((2 * 16 * L,), jnp.int32),
            pltpu.VMEM((2 * 16 * L,), jnp.int32),
            pltpu.VMEM((G_MAX * L,), jnp.float32),
            pltpu.SemaphoreType.DMA,
            pltpu.SemaphoreType.DMA,
        ],
    )


def kernel(x, y, SoS):
    thetas = jnp.linspace(0.0, 2.0 * jnp.pi, N_POINTS, dtype=jnp.float32)
    r = jnp.sqrt(x ** 2 + y ** 2)
    phi = jnp.arctan2(x, y)
    t = thetas - phi
    chord = jnp.sqrt(R_BODY ** 2 - (r * jnp.sin(t)) ** 2)
    l = chord + r * jnp.cos(t)

    ax = (x[0] - X0) / DX + 0.5
    ay = -((y[0] - Y0) / DY + 0.5)
    bx = -(l * jnp.sin(thetas)) / jnp.float32(DX * (N_INT - 1))
    by = (l * jnp.cos(thetas)) / jnp.float32(DY * (N_INT - 1))
    scale = l / jnp.float32(2 * (N_INT - 1))
    tbl = (1.0 - V0 / SoS).astype(jnp.float32).reshape(-1)

    pad = N_GROUPS_PAD * L - N_POINTS
    params = jnp.stack([
        jnp.pad(bx, (0, pad)).reshape(N_GROUPS_PAD, L),
        jnp.pad(by, (0, pad)).reshape(N_GROUPS_PAD, L),
        jnp.pad(scale, (0, pad)).reshape(N_GROUPS_PAD, L),
    ], axis=1).reshape(-1)

    marange = jnp.arange(17, dtype=jnp.float32)
    nzmask = tbl != 0.0
    order = jnp.argsort(jnp.where(nzmask, 0, 1), stable=True).astype(jnp.int32)
    xoff = (order & 15) * L
    yoff = ((((order >> 4) + 15) & 15) + 16) * L - 16 * L
    packed = xoff | (yoff << 8) | ((order * L) << 16)
    consts = jnp.concatenate([
        jnp.repeat(tbl, L),
        jnp.full((L,), ax, dtype=jnp.float32),
        jnp.full((L,), ay, dtype=jnp.float32),
        jnp.repeat((marange - ax).astype(jnp.float32), L),
        jnp.repeat((marange - 16.0 - ay).astype(jnp.float32), L),
        tbl,
        jnp.full((L,), jnp.sum(nzmask), dtype=jnp.float32),
        jax.lax.bitcast_convert_type(packed, jnp.float32),
    ])

    wf = _sc_integrate()(jnp.concatenate([params, consts]))
    return thetas, wf


# speedup vs baseline: 1.0719x; 1.0719x over previous
import functools

import jax
import jax.numpy as jnp
from jax import lax
from jax.experimental import pallas as pl
from jax.experimental.pallas import tpu as pltpu
from jax.experimental.pallas import tpu_sc as plsc

N_POINTS = 3600
N_INT = 2000
R_BODY = 10.0
V0 = 1540.0
X0 = -12.0
DX = 1.6
Y0 = -12.0
DY = 1.6

L = 16
NC = 2
NS = 16
NW = NC * NS
N_GROUPS = N_POINTS // L
N_GROUPS_PAD = N_GROUPS + 1
G_MAX = 8
EPS = 1e-12

O_AX = 4096
O_AY = 4112
O_CX = 4128
O_CY = 4400
O_TBL = 4672
CONSTS_LEN = 4928
PARAMS_LEN = (N_GROUPS + 1) * 3 * 16


def _sc_body(in_hbm, wf_hbm, consts_v, params_v, yint_v, out_v, sem1, sem2):
    wid = lax.axis_index("s") * NC + lax.axis_index("c")
    start_g = 7 * wid + jnp.minimum(wid, 1)
    cp1 = pltpu.async_copy(in_hbm.at[pl.ds(PARAMS_LEN, CONSTS_LEN)], consts_v,
                           sem1)
    cp2 = pltpu.async_copy(
        in_hbm.at[pl.ds(start_g * (3 * L), G_MAX * 3 * L)], params_v, sem2)
    cp1.wait()
    cp2.wait()
    tbl_v = consts_v.at[pl.ds(O_TBL, 256)]
    axv = consts_v[pl.ds(O_AX, L)]
    ayv = consts_v[pl.ds(O_AY, L)]
    iota = jax.lax.iota(jnp.int32, L)
    n_w = jnp.where(wid == 0, G_MAX, 7)

    def gather_at(xf, yf):
        xi = xf.astype(jnp.int32)
        row = yf.astype(jnp.int32) & 15
        return plsc.load_gather(tbl_v, [(row << 4) + xi])

    def run_group(i, _):
        off = i * (3 * L)
        bx = plsc.load_gather(params_v, [off + iota])
        by = plsc.load_gather(params_v, [off + L + iota])
        sc = plsc.load_gather(params_v, [off + 2 * L + iota])

        def boundaries(b, c_base):
            bp = jnp.where(b >= 0, jnp.maximum(b, EPS), jnp.minimum(b, -EPS))
            inv = 1.0 / bp
            bpos = bp > 0.0

            def q_at(m):
                t = consts_v[pl.ds(c_base + m * L, L)] * inv
                t = jnp.clip(t, -1.0, float(N_INT + 1))
                ti = t.astype(jnp.int32)
                tf = ti.astype(jnp.float32)
                up = jnp.where(jnp.where(bpos, t > tf, t >= tf), 1, 0)
                return jnp.clip(ti + up, 0, N_INT)

            q_prev = q_at(0)
            los, hips = [], []
            for m in range(1, 17):
                q_cur = q_at(m)
                los.append(jnp.minimum(q_prev, q_cur))
                hips.append(jnp.maximum(q_prev, q_cur))
                q_prev = q_cur
            return los, hips

        ylos, yhips = boundaries(by, O_CY)
        for k in range(16):
            yint_v[pl.ds(k * L, L)] = ylos[k]
            yint_v[pl.ds((16 + k) * L, L)] = yhips[k]
        xlos, xhips = boundaries(bx, O_CX)

        accs = [jnp.zeros((L,), jnp.float32) for _ in range(4)]
        for k in range(16):
            ylo = yint_v[pl.ds(k * L, L)]
            yhip = yint_v[pl.ds((16 + k) * L, L)]
            row = (k + 1) & 15
            for m in range(16):
                cnt = jnp.minimum(xhips[m], yhip) - jnp.maximum(xlos[m], ylo)
                cnt = jnp.maximum(cnt, 0).astype(jnp.float32)
                tv = consts_v[pl.ds((row * 16 + m) * L, L)]
                accs[m % 4] = accs[m % 4] + cnt * tv
        acc = (accs[0] + accs[1]) + (accs[2] + accs[3])

        v_first = gather_at(axv, ayv)
        v_last = gather_at(axv + bx * float(N_INT - 1),
                           ayv + by * float(N_INT - 1))
        wf = sc * (2.0 * acc - v_first - v_last)
        plsc.store_scatter(out_v, [i * L + iota], wf)
        return 0

    lax.fori_loop(0, n_w, run_group, 0)

    @pl.when(wid == 0)
    def _():
        pltpu.sync_copy(out_v, wf_hbm.at[pl.ds(0, G_MAX * L)])

    @pl.when(wid > 0)
    def _():
        pltpu.sync_copy(out_v.at[pl.ds(0, 7 * L)],
                        wf_hbm.at[pl.ds(G_MAX * L + (wid - 1) * 7 * L, 7 * L)])


@functools.cache
def _sc_integrate():
    return pl.kernel(
        _sc_body,
        out_type=jax.ShapeDtypeStruct((N_POINTS,), jnp.float32),
        mesh=plsc.VectorSubcoreMesh(core_axis_name="c", subcore_axis_name="s"),
        compiler_params=pltpu.CompilerParams(needs_layout_passes=False),
        scratch_types=[
            pltpu.VMEM((CONSTS_LEN,), jnp.float32),
            pltpu.VMEM((G_MAX * 3 * L,), jnp.float32),
            pltpu.VMEM((2 * 16 * L,), jnp.int32),
            pltpu.VMEM((G_MAX * L,), jnp.float32),
            pltpu.SemaphoreType.DMA,
            pltpu.SemaphoreType.DMA,
        ],
    )


def kernel(x, y, SoS):
    thetas = jnp.linspace(0.0, 2.0 * jnp.pi, N_POINTS, dtype=jnp.float32)
    r = jnp.sqrt(x ** 2 + y ** 2)
    phi = jnp.arctan2(x, y)
    t = thetas - phi
    chord = jnp.sqrt(R_BODY ** 2 - (r * jnp.sin(t)) ** 2)
    l = chord + r * jnp.cos(t)

    ax = (x[0] - X0) / DX + 0.5
    ay = -((y[0] - Y0) / DY + 0.5)
    bx = -(l * jnp.sin(thetas)) / jnp.float32(DX * (N_INT - 1))
    by = (l * jnp.cos(thetas)) / jnp.float32(DY * (N_INT - 1))
    scale = l / jnp.float32(2 * (N_INT - 1))
    tbl = (1.0 - V0 / SoS).astype(jnp.float32).reshape(-1)

    pad = N_GROUPS_PAD * L - N_POINTS
    params = jnp.stack([
        jnp.pad(bx, (0, pad)).reshape(N_GROUPS_PAD, L),
        jnp.pad(by, (0, pad)).reshape(N_GROUPS_PAD, L),
        jnp.pad(scale, (0, pad)).reshape(N_GROUPS_PAD, L),
    ], axis=1).reshape(-1)

    marange = jnp.arange(17, dtype=jnp.float32)
    consts = jnp.concatenate([
        jnp.repeat(tbl, L),
        jnp.full((L,), ax, dtype=jnp.float32),
        jnp.full((L,), ay, dtype=jnp.float32),
        jnp.repeat((marange - ax).astype(jnp.float32), L),
        jnp.repeat((marange - 16.0 - ay).astype(jnp.float32), L),
        tbl,
    ])

    wf = _sc_integrate()(jnp.concatenate([params, consts]))
    return thetas, wf
